# baseline (speedup 1.0000x reference)
import functools

import jax
import jax.numpy as jnp
from jax import lax
from jax.experimental import pallas as pl
from jax.experimental.pallas import tpu as pltpu
from jax.experimental.pallas import tpu_sc as plsc

_NC = 2
_NS = 16
_NW = _NC * _NS
_CHUNK = 128


def _gather_body(nchunk, idx_hbm, table_hbm, out_hbm, idx_v, rows_v, sem):
    wid = lax.axis_index("s") * _NC + lax.axis_index("c")
    bpw = nchunk * _CHUNK
    base = wid * bpw
    pltpu.sync_copy(idx_hbm.at[wid], idx_v)
    copies = []
    for j in range(nchunk):
        copies.append(
            pltpu.async_copy(
                table_hbm.at[idx_v.at[j]],
                rows_v.at[pl.ds(j * _CHUNK, _CHUNK)],
                sem,
            )
        )
    for c in copies:
        c.wait()
    pltpu.sync_copy(rows_v, out_hbm.at[pl.ds(base, bpw)])


def kernel(user_inputs, userEmbedding_weight):
    batch = user_inputs.shape[0]
    embed_dim = userEmbedding_weight.shape[1]
    bpw = batch // _NW
    nchunk = bpw // _CHUNK
    idx = user_inputs.astype(jnp.int32).reshape(_NW, nchunk, _CHUNK)
    mesh = plsc.VectorSubcoreMesh(core_axis_name="c", subcore_axis_name="s")
    f = pl.kernel(
        functools.partial(_gather_body, nchunk),
        out_type=jax.ShapeDtypeStruct((batch, embed_dim), jnp.float32),
        mesh=mesh,
        compiler_params=pltpu.CompilerParams(use_tc_tiling_on_sc=False),
        scratch_types=[
            pltpu.VMEM((nchunk, _CHUNK), jnp.int32),
            pltpu.VMEM((bpw, embed_dim), jnp.float32),
            pltpu.SemaphoreType.DMA,
        ],
    )
    return f(idx, userEmbedding_weight)


# speedup vs baseline: 1.1203x; 1.1203x over previous
import jax
import jax.numpy as jnp
from jax import lax
from jax.experimental import pallas as pl
from jax.experimental.pallas import tpu as pltpu
from jax.experimental.pallas import tpu_sc as plsc

_NC = 2
_NS = 16
_NW = _NC * _NS
_D = 64
_DP = 128
_B = 16384
_BPW = _B // _NW
_CHUNK = 128
_NCHUNK = _BPW // _CHUNK


def _gather_body(idx_hbm, tab_hbm, out_hbm, idx_v, rows_v, sem):
    wid = lax.axis_index("s") * _NC + lax.axis_index("c")
    base = wid * _BPW
    pltpu.sync_copy(idx_hbm.at[wid], idx_v)
    copies = []
    for j in range(_NCHUNK):
        copies.append(
            pltpu.async_copy(
                tab_hbm.at[idx_v.at[j]],
                rows_v.at[pl.ds(j * _CHUNK, _CHUNK)],
                sem,
            )
        )
    for c in copies:
        c.wait()
    pltpu.sync_copy(rows_v, out_hbm.at[pl.ds(base, _BPW)])


def kernel(user_inputs, userEmbedding_weight):
    idx = user_inputs.astype(jnp.int32).reshape(_NW, _NCHUNK, _CHUNK)
    tab = jnp.pad(userEmbedding_weight, ((0, 0), (0, _DP - _D)))
    mesh = plsc.VectorSubcoreMesh(core_axis_name="c", subcore_axis_name="s")
    f = pl.kernel(
        _gather_body,
        out_type=jax.ShapeDtypeStruct((_B, _DP), jnp.float32),
        mesh=mesh,
        scratch_types=[
            pltpu.VMEM((_NCHUNK, _CHUNK), jnp.int32),
            pltpu.VMEM((_BPW, _DP), jnp.float32),
            pltpu.SemaphoreType.DMA,
        ],
    )
    out128 = f(idx, tab)
    return out128[:, :_D]


# speedup vs baseline: 2.7728x; 2.4750x over previous
import jax
import jax.numpy as jnp
from jax import lax
from jax.experimental import pallas as pl
from jax.experimental.pallas import tpu as pltpu
from jax.experimental.pallas import tpu_sc as plsc

_NC = 2
_NS = 16
_NW = _NC * _NS
_D = 64
_DP = 128
_B = 16384
_V = 1_000_000
_SLABW = 256
_NSLAB = _V // _SLABW
_TAIL0 = _NSLAB * _SLABW
_SPT = -(-_NSLAB // _NW)
_SAC = _B


def _splat(x):
    return jnp.full((16,), x, jnp.int32)


def _scal(v):
    return v[0]


def _body(idx_hbm, tabT_hbm, tail_hbm, out_hbm,
          idx_v, lane_l, pos_l, slab_v, stage_v, clane, cpos2,
          sem_slab, sem_sc):
    wid = lax.axis_index("s") * _NC + lax.axis_index("c")
    sac = _SAC + wid
    slab0 = wid * _SPT
    ns = jnp.minimum(_SPT, _NSLAB - slab0)
    lo = slab0 * _SLABW
    hi = lo + ns * _SLABW
    hi_f = jnp.where(wid == _NW - 1, _V, hi)

    pltpu.sync_copy(idx_hbm, idx_v)
    iota = lax.iota(jnp.int32, 16)

    def p1(g, ptr):
        r = g >> 3
        col = (g & 7) * 16
        vec = idx_v[r, pl.ds(col, 16)]
        m = (vec >= lo) & (vec < hi_f)
        mi = m.astype(jnp.int32)
        dst = ptr + plsc.cumsum(mi) - mi
        plsc.store_scatter(lane_l, [dst], vec, mask=m)
        plsc.store_scatter(pos_l, [dst], _splat(g * 16) + iota, mask=m)
        return ptr + plsc.all_reduce_population_count(m)

    ptr = lax.fori_loop(0, (_B // 16), p1, _splat(0))
    nl = _scal(ptr)

    def slab_src(s):
        c0 = pl.multiple_of((slab0 + s) * _SLABW, 128)
        return tabT_hbm.at[:, pl.ds(c0, _SLABW)]

    @pl.when(ns > 0)
    def _():
        pltpu.async_copy(slab_src(0), slab_v.at[0], sem_slab)

    def flush(c):
        slot16 = pl.multiple_of(16 * lax.rem(c, 16), 16)
        pltpu.async_copy(
            stage_v.at[pl.ds(slot16, 16)], out_hbm.at[cpos2.at[0]], sem_sc
        ).wait()
        cpos2[0, pl.ds(0, 16)] = cpos2[1, pl.ds(0, 16)]
        clane[pl.ds(0, 16)] = clane[pl.ds(16, 16)]

    def process(c0, hi_s, sp, carry):
        ng = (nl + 15) >> 4

        def grp(g, carry):
            cs, c = carry
            lv = lane_l[pl.ds(16 * g, 16)]
            pv = pos_l[pl.ds(16 * g, 16)]
            m = (lv >= c0) & (lv < hi_s) & (iota + 16 * g < nl)
            mi = m.astype(jnp.int32)
            kv = _splat(cs) + plsc.cumsum(mi) - mi
            plsc.store_scatter(clane, [kv], lv - c0, mask=m)
            plsc.store_scatter(cpos2, [kv >> 4, kv & 15], pv, mask=m)
            cnt = _scal(plsc.all_reduce_population_count(m))

            def hit(k, _):
                lane16 = plsc.load_gather(clane, [_splat(k)])
                ovf = k >= 16
                srow = 16 * lax.rem(c + ovf.astype(jnp.int32), 16) \
                    + lax.rem(k, 16)
                for r in range(4):
                    g16 = plsc.load_gather(
                        slab_v, [_splat(sp), iota + 16 * r, lane16]
                    )
                    stage_v[srow, pl.ds(16 * r, 16)] = g16
                return 0

            lax.fori_loop(cs, cs + cnt, hit, 0)
            ncs = cs + cnt
            full = ncs >= 16

            @pl.when(full)
            def _():
                flush(c)

            return (jnp.where(full, ncs - 16, ncs),
                    jnp.where(full, c + 1, c))

        return lax.fori_loop(0, ng, grp, carry)

    def slab_step(s, carry):
        @pl.when(s + 1 < ns)
        def _():
            pltpu.async_copy(slab_src(s + 1), slab_v.at[lax.rem(s + 1, 2)],
                             sem_slab)
        pltpu.make_async_copy(slab_src(s), slab_v.at[lax.rem(s, 2)],
                              sem_slab).wait()
        c0 = (slab0 + s) * _SLABW
        return process(c0, c0 + _SLABW, lax.rem(s, 2), carry)

    carry = lax.fori_loop(0, ns, slab_step, (jnp.int32(0), jnp.int32(0)))

    pltpu.sync_copy(tail_hbm, slab_v.at[0, :, pl.ds(0, _DP)])
    carry = process(jnp.int32(_TAIL0), jnp.int32(_V), jnp.int32(0), carry)
    cs, c = carry

    @pl.when(cs > 0)
    def _():
        row = cpos2[0, pl.ds(0, 16)]
        cpos2[0, pl.ds(0, 16)] = jnp.where(iota < cs, row, _splat(sac))
        flush(c)


def kernel(user_inputs, userEmbedding_weight):
    idx = user_inputs.astype(jnp.int32).reshape(128, 128)
    tabT = userEmbedding_weight.T
    tail = jnp.pad(userEmbedding_weight[_TAIL0:].T, ((0, 0), (0, _DP - _D)))
    mesh = plsc.VectorSubcoreMesh(core_axis_name="c", subcore_axis_name="s")
    f = pl.kernel(
        _body,
        out_type=jax.ShapeDtypeStruct((_B + 128, _DP), jnp.float32),
        mesh=mesh,
        compiler_params=pltpu.CompilerParams(needs_layout_passes=False),
        scratch_types=[
            pltpu.VMEM((128, 128), jnp.int32),
            pltpu.VMEM((_B,), jnp.int32),
            pltpu.VMEM((_B,), jnp.int32),
            pltpu.VMEM((2, _D, _SLABW), jnp.float32),
            pltpu.VMEM((256, _DP), jnp.float32),
            pltpu.VMEM((32,), jnp.int32),
            pltpu.VMEM((2, 16), jnp.int32),
            pltpu.SemaphoreType.DMA,
            pltpu.SemaphoreType.DMA,
        ],
    )
    outp = f(idx, tabT, tail)
    return outp[:_B, :_D]
